# Initial kernel scaffold; baseline (speedup 1.0000x reference)
import jax
import jax.numpy as jnp
from jax import lax
from jax.experimental import pallas as pl
from jax.experimental.pallas import tpu as pltpu
from jax.experimental.pallas import tpu_sc as plsc

N = 10000
E = 320000
DF = 128
DH = 16
NCLS = 7

NPAD = 10240
RPT = NPAD // 16
NW = 32
EPT = 10240
EB = EPT // 128
KB = 8
NCHUNK = EB // KB

_mesh = plsc.VectorSubcoreMesh(core_axis_name="c", subcore_axis_name="s")


def _make_agg(width):
    def body(y_hbm, src_hbm, dst_hbm, zero_hbm, out_hbm,
             src_v, dst_v, rows_v, agg_sh, sem):
        c = lax.axis_index("c")
        s = lax.axis_index("s")
        wid = c * 16 + s
        pltpu.sync_copy(zero_hbm, agg_sh.at[pl.ds(s * RPT, RPT)])
        pltpu.sync_copy(src_hbm.at[wid], src_v)
        pltpu.sync_copy(dst_hbm.at[wid], dst_v)
        plsc.subcore_barrier()

        def chunk(k, carry):
            descs = []
            for b in range(KB):
                d = pltpu.async_copy(
                    y_hbm.at[src_v.at[k * KB + b]], rows_v.at[b], sem)
                descs.append(d)
            for d in descs:
                d.wait()
            for b in range(KB):
                pltpu.sync_copy(rows_v.at[b],
                                agg_sh.at[dst_v.at[k * KB + b]], add=True)
            return carry

        lax.fori_loop(0, NCHUNK, chunk, 0)
        plsc.subcore_barrier()
        pltpu.sync_copy(agg_sh.at[pl.ds(s * RPT, RPT)],
                        out_hbm.at[c].at[pl.ds(s * RPT, RPT)])

    return pl.kernel(
        body,
        out_type=jax.ShapeDtypeStruct((2, NPAD, width), jnp.float32),
        mesh=_mesh,
        scratch_types=[
            pltpu.VMEM((EB, 128), jnp.int32),
            pltpu.VMEM((EB, 128), jnp.int32),
            pltpu.VMEM((KB, 128, width), jnp.float32),
            pltpu.VMEM_SHARED((NPAD, width), jnp.float32),
            pltpu.SemaphoreType.DMA,
        ],
    )


def _deg_body(dst_hbm, zero_hbm, out_hbm, dst_v, ones_v, deg_sh, sem):
    c = lax.axis_index("c")
    s = lax.axis_index("s")
    wid = c * 16 + s
    pltpu.sync_copy(zero_hbm, deg_sh.at[pl.ds(s * RPT, RPT)])
    pltpu.sync_copy(dst_hbm.at[wid], dst_v)
    for i in range(8):
        ones_v[pl.ds(16 * i, 16)] = jnp.ones((16,), jnp.float32)
    plsc.subcore_barrier()

    def chunk(k, carry):
        pltpu.sync_copy(ones_v, deg_sh.at[dst_v.at[k]], add=True)
        return carry

    lax.fori_loop(0, EB, chunk, 0)
    plsc.subcore_barrier()
    pltpu.sync_copy(deg_sh.at[pl.ds(s * RPT, RPT)],
                    out_hbm.at[c].at[pl.ds(s * RPT, RPT)])


_deg_call = pl.kernel(
    _deg_body,
    out_type=jax.ShapeDtypeStruct((2, NPAD), jnp.float32),
    mesh=_mesh,
    scratch_types=[
        pltpu.VMEM((EB, 128), jnp.int32),
        pltpu.VMEM((128,), jnp.float32),
        pltpu.VMEM_SHARED((NPAD,), jnp.float32),
        pltpu.SemaphoreType.DMA,
    ],
)

_GRID = 8
_BR = NPAD // _GRID


def _tc1_body(degp_ref, x_ref, w1_ref, y1_ref, dinv_ref):
    deg = degp_ref[:, 0:1] + degp_ref[:, 1:2] + 1.0
    dinv = lax.rsqrt(deg)
    xw = jnp.dot(x_ref[...], w1_ref[...], preferred_element_type=jnp.float32)
    y1_ref[...] = xw * dinv
    dinv_ref[...] = dinv


def _tc2_body(agg_ref, y1_ref, dinv_ref, b1_ref, w2_ref, y2_ref):
    h = (agg_ref[0] + agg_ref[1] + y1_ref[...]) * dinv_ref[...] + b1_ref[...]
    h = jnp.maximum(h, 0.0)
    y2_ref[...] = jnp.dot(h, w2_ref[...],
                          preferred_element_type=jnp.float32) * dinv_ref[...]


def _tc3_body(agg_ref, y2_ref, dinv_ref, b2_ref, out_ref):
    z = (agg_ref[0] + agg_ref[1] + y2_ref[...]) * dinv_ref[...] + b2_ref[...]
    z = jnp.maximum(z, 0.0)
    col = lax.broadcasted_iota(jnp.int32, z.shape, 1)
    zm = jnp.where(col < NCLS, z, -jnp.inf)
    m = jnp.max(zm, axis=1, keepdims=True)
    e = jnp.where(col < NCLS, jnp.exp(z - m), 0.0)
    lse = jnp.log(jnp.sum(e, axis=1, keepdims=True))
    out_ref[...] = z - m - lse


def _row_spec(width):
    return pl.BlockSpec((_BR, width), lambda i: (i, 0))


def _agg_spec(width):
    return pl.BlockSpec((2, _BR, width), lambda i: (0, i, 0))


_tc1_call = pl.pallas_call(
    _tc1_body,
    grid=(_GRID,),
    in_specs=[_row_spec(2),
              _row_spec(DF),
              pl.BlockSpec((DF, DH), lambda i: (0, 0))],
    out_specs=[_row_spec(DH), _row_spec(1)],
    out_shape=[jax.ShapeDtypeStruct((NPAD, DH), jnp.float32),
               jax.ShapeDtypeStruct((NPAD, 1), jnp.float32)],
)

_tc2_call = pl.pallas_call(
    _tc2_body,
    grid=(_GRID,),
    in_specs=[_agg_spec(DH),
              _row_spec(DH),
              _row_spec(1),
              pl.BlockSpec((1, DH), lambda i: (0, 0)),
              pl.BlockSpec((DH, 8), lambda i: (0, 0))],
    out_specs=_row_spec(8),
    out_shape=jax.ShapeDtypeStruct((NPAD, 8), jnp.float32),
)

_tc3_call = pl.pallas_call(
    _tc3_body,
    grid=(_GRID,),
    in_specs=[_agg_spec(8),
              _row_spec(8),
              _row_spec(1),
              pl.BlockSpec((1, 8), lambda i: (0, 0))],
    out_specs=_row_spec(8),
    out_shape=jax.ShapeDtypeStruct((NPAD, 8), jnp.float32),
)

_agg_call16 = _make_agg(DH)
_agg_call8 = _make_agg(8)


@jax.jit
def kernel(x, edge_index, W1, b1, W2, b2):
    epad = jnp.full((NW * EPT - E,), N, dtype=jnp.int32)
    srcp = jnp.concatenate([edge_index[0], epad]).reshape(NW, EB, 128)
    dstp = jnp.concatenate([edge_index[1], epad]).reshape(NW, EB, 128)
    x_pad = jnp.concatenate(
        [x, jnp.zeros((NPAD - N, DF), jnp.float32)], axis=0)
    w2p = jnp.concatenate([W2, jnp.zeros((DH, 1), jnp.float32)], axis=1)
    b2p = jnp.concatenate([b2, jnp.zeros((1,), jnp.float32)]).reshape(1, 8)

    degp = _deg_call(dstp, jnp.zeros((RPT,), jnp.float32))
    y1, dinv = _tc1_call(degp.T, x_pad, W1)
    agg1 = _agg_call16(y1, srcp, dstp, jnp.zeros((RPT, DH), jnp.float32))
    y2 = _tc2_call(agg1, y1, dinv, b1.reshape(1, DH), w2p)
    agg2 = _agg_call8(y2, srcp, dstp, jnp.zeros((RPT, 8), jnp.float32))
    out = _tc3_call(agg2, y2, dinv, b2p)
    return out[:N, :NCLS]


# speedup vs baseline: 36.6931x; 36.6931x over previous
import jax
import jax.numpy as jnp
from jax import lax
from jax.experimental import pallas as pl
from jax.experimental.pallas import tpu as pltpu
from jax.experimental.pallas import tpu_sc as plsc

N = 10000
E = 320000
DF = 128
DH = 16
NCLS = 7

NPAD = 10240
RPT = NPAD // 16
NW = 32
EPT = 10240
EB = EPT // 128
KB = 8
NCHUNK = EB // KB

_mesh = plsc.VectorSubcoreMesh(core_axis_name="c", subcore_axis_name="s")


def _make_agg(width):
    def body(y_hbm, src_hbm, dst_hbm, zero_hbm, out_hbm,
             src_v, dst_v, rows_v, agg_sh, sem):
        c = lax.axis_index("c")
        s = lax.axis_index("s")
        wid = c * 16 + s
        pltpu.sync_copy(zero_hbm, agg_sh.at[pl.ds(s * RPT, RPT)])
        pltpu.sync_copy(src_hbm.at[wid], src_v)
        pltpu.sync_copy(dst_hbm.at[wid], dst_v)
        plsc.subcore_barrier()

        def chunk(k, carry):
            descs = []
            for b in range(KB):
                d = pltpu.async_copy(
                    y_hbm.at[src_v.at[k * KB + b]], rows_v.at[b], sem)
                descs.append(d)
            for d in descs:
                d.wait()
            for b in range(KB):
                pltpu.sync_copy(rows_v.at[b],
                                agg_sh.at[dst_v.at[k * KB + b]], add=True)
            return carry

        lax.fori_loop(0, NCHUNK, chunk, 0)
        plsc.subcore_barrier()
        pltpu.sync_copy(agg_sh.at[pl.ds(s * RPT, RPT)],
                        out_hbm.at[c].at[pl.ds(s * RPT, RPT)])

    return pl.kernel(
        body,
        out_type=jax.ShapeDtypeStruct((2, NPAD, width), jnp.float32),
        mesh=_mesh,
        compiler_params=pltpu.CompilerParams(use_tc_tiling_on_sc=False),
        scratch_types=[
            pltpu.VMEM((EB, 128), jnp.int32),
            pltpu.VMEM((EB, 128), jnp.int32),
            pltpu.VMEM((KB, 128, width), jnp.float32),
            pltpu.VMEM_SHARED((NPAD, width), jnp.float32),
            pltpu.SemaphoreType.DMA,
        ],
    )


def _deg_body(dst_hbm, zero_hbm, out_hbm, dst_v, ones_v, deg_sh, sem):
    c = lax.axis_index("c")
    s = lax.axis_index("s")
    wid = c * 16 + s
    pltpu.sync_copy(zero_hbm, deg_sh.at[pl.ds(s * RPT, RPT)])
    pltpu.sync_copy(dst_hbm.at[wid], dst_v)
    for i in range(8):
        ones_v[pl.ds(16 * i, 16)] = jnp.ones((16,), jnp.float32)
    plsc.subcore_barrier()

    def chunk(k, carry):
        pltpu.sync_copy(ones_v, deg_sh.at[dst_v.at[k]], add=True)
        return carry

    lax.fori_loop(0, EB, chunk, 0)
    plsc.subcore_barrier()
    pltpu.sync_copy(deg_sh.at[pl.ds(s * RPT, RPT)],
                    out_hbm.at[c].at[pl.ds(s * RPT, RPT)])


_deg_call = pl.kernel(
    _deg_body,
    out_type=jax.ShapeDtypeStruct((2, NPAD), jnp.float32),
    mesh=_mesh,
    compiler_params=pltpu.CompilerParams(use_tc_tiling_on_sc=False),
    scratch_types=[
        pltpu.VMEM((EB, 128), jnp.int32),
        pltpu.VMEM((128,), jnp.float32),
        pltpu.VMEM_SHARED((NPAD,), jnp.float32),
        pltpu.SemaphoreType.DMA,
    ],
)

_GRID = 8
_BR = NPAD // _GRID


def _tc1_body(degp_ref, x_ref, w1_ref, y1_ref, dinv_ref):
    deg = degp_ref[:, 0:1] + degp_ref[:, 1:2] + 1.0
    dinv = lax.rsqrt(deg)
    xw = jnp.dot(x_ref[...], w1_ref[...], preferred_element_type=jnp.float32)
    y1_ref[...] = xw * dinv
    dinv_ref[...] = dinv


def _tc2_body(agg_ref, y1_ref, dinv_ref, b1_ref, w2_ref, y2_ref):
    h = (agg_ref[0] + agg_ref[1] + y1_ref[...]) * dinv_ref[...] + b1_ref[...]
    h = jnp.maximum(h, 0.0)
    y2_ref[...] = jnp.dot(h, w2_ref[...],
                          preferred_element_type=jnp.float32) * dinv_ref[...]


def _tc3_body(agg_ref, y2_ref, dinv_ref, b2_ref, out_ref):
    z = (agg_ref[0] + agg_ref[1] + y2_ref[...]) * dinv_ref[...] + b2_ref[...]
    z = jnp.maximum(z, 0.0)
    col = lax.broadcasted_iota(jnp.int32, z.shape, 1)
    zm = jnp.where(col < NCLS, z, -jnp.inf)
    m = jnp.max(zm, axis=1, keepdims=True)
    e = jnp.where(col < NCLS, jnp.exp(z - m), 0.0)
    lse = jnp.log(jnp.sum(e, axis=1, keepdims=True))
    out_ref[...] = z - m - lse


def _row_spec(width):
    return pl.BlockSpec((_BR, width), lambda i: (i, 0))


def _agg_spec(width):
    return pl.BlockSpec((2, _BR, width), lambda i: (0, i, 0))


_tc1_call = pl.pallas_call(
    _tc1_body,
    grid=(_GRID,),
    in_specs=[_row_spec(2),
              _row_spec(DF),
              pl.BlockSpec((DF, DH), lambda i: (0, 0))],
    out_specs=[_row_spec(DH), _row_spec(1)],
    out_shape=[jax.ShapeDtypeStruct((NPAD, DH), jnp.float32),
               jax.ShapeDtypeStruct((NPAD, 1), jnp.float32)],
)

_tc2_call = pl.pallas_call(
    _tc2_body,
    grid=(_GRID,),
    in_specs=[_agg_spec(DH),
              _row_spec(DH),
              _row_spec(1),
              pl.BlockSpec((1, DH), lambda i: (0, 0)),
              pl.BlockSpec((DH, 8), lambda i: (0, 0))],
    out_specs=_row_spec(8),
    out_shape=jax.ShapeDtypeStruct((NPAD, 8), jnp.float32),
)

_tc3_call = pl.pallas_call(
    _tc3_body,
    grid=(_GRID,),
    in_specs=[_agg_spec(8),
              _row_spec(8),
              _row_spec(1),
              pl.BlockSpec((1, 8), lambda i: (0, 0))],
    out_specs=_row_spec(8),
    out_shape=jax.ShapeDtypeStruct((NPAD, 8), jnp.float32),
)

_agg_call16 = _make_agg(DH)
_agg_call8 = _make_agg(8)


@jax.jit
def kernel(x, edge_index, W1, b1, W2, b2):
    epad = jnp.full((NW * EPT - E,), N, dtype=jnp.int32)
    srcp = jnp.concatenate([edge_index[0], epad]).reshape(NW, EB, 128)
    dstp = jnp.concatenate([edge_index[1], epad]).reshape(NW, EB, 128)
    x_pad = jnp.concatenate(
        [x, jnp.zeros((NPAD - N, DF), jnp.float32)], axis=0)
    w2p = jnp.concatenate([W2, jnp.zeros((DH, 1), jnp.float32)], axis=1)
    b2p = jnp.concatenate([b2, jnp.zeros((1,), jnp.float32)]).reshape(1, 8)

    degp = _deg_call(dstp, jnp.zeros((RPT,), jnp.float32))
    y1, dinv = _tc1_call(degp.T, x_pad, W1)
    agg1 = _agg_call16(y1, srcp, dstp, jnp.zeros((RPT, DH), jnp.float32))
    y2 = _tc2_call(agg1, y1, dinv, b1.reshape(1, DH), w2p)
    agg2 = _agg_call8(y2, srcp, dstp, jnp.zeros((RPT, 8), jnp.float32))
    out = _tc3_call(agg2, y2, dinv, b2p)
    return out[:N, :NCLS]


# speedup vs baseline: 37.2414x; 1.0149x over previous
import jax
import jax.numpy as jnp
from jax import lax
from jax.experimental import pallas as pl
from jax.experimental.pallas import tpu as pltpu
from jax.experimental.pallas import tpu_sc as plsc

N = 10000
E = 320000
DF = 128
DH = 16
NCLS = 7

NPAD = 10240
RPT = NPAD // 16
NW = 32
EPT = 10240
EB = EPT // 128
KB = 8
NPAIR = EB // (2 * KB)

_mesh = plsc.VectorSubcoreMesh(core_axis_name="c", subcore_axis_name="s")


def _make_agg(width):
    def body(y_hbm, src_hbm, dst_hbm, zero_hbm, out_hbm,
             src_v, dst_v, rows0_v, rows1_v, agg_sh,
             sem_g0, sem_g1, sem_s):
        c = lax.axis_index("c")
        s = lax.axis_index("s")
        wid = c * 16 + s
        pltpu.sync_copy(zero_hbm, agg_sh.at[pl.ds(s * RPT, RPT)])
        pltpu.sync_copy(src_hbm.at[wid], src_v)
        pltpu.sync_copy(dst_hbm.at[wid], dst_v)
        plsc.subcore_barrier()

        def pair(k, carry):
            e0 = (2 * k) * KB
            e1 = (2 * k + 1) * KB
            g0 = [pltpu.async_copy(y_hbm.at[src_v.at[e0 + b]],
                                   rows0_v.at[b], sem_g0)
                  for b in range(KB)]
            g1 = [pltpu.async_copy(y_hbm.at[src_v.at[e1 + b]],
                                   rows1_v.at[b], sem_g1)
                  for b in range(KB)]
            for d in g0:
                d.wait()
            s0 = [pltpu.async_copy(rows0_v.at[b],
                                   agg_sh.at[dst_v.at[e0 + b]],
                                   sem_s, add=True)
                  for b in range(KB)]
            for d in g1:
                d.wait()
            s1 = [pltpu.async_copy(rows1_v.at[b],
                                   agg_sh.at[dst_v.at[e1 + b]],
                                   sem_s, add=True)
                  for b in range(KB)]
            for d in s0 + s1:
                d.wait()
            return carry

        lax.fori_loop(0, NPAIR, pair, 0)
        plsc.subcore_barrier()
        pltpu.sync_copy(agg_sh.at[pl.ds(s * RPT, RPT)],
                        out_hbm.at[c].at[pl.ds(s * RPT, RPT)])

    return pl.kernel(
        body,
        out_type=jax.ShapeDtypeStruct((2, NPAD, width), jnp.float32),
        mesh=_mesh,
        compiler_params=pltpu.CompilerParams(use_tc_tiling_on_sc=False),
        scratch_types=[
            pltpu.VMEM((EB, 128), jnp.int32),
            pltpu.VMEM((EB, 128), jnp.int32),
            pltpu.VMEM((KB, 128, width), jnp.float32),
            pltpu.VMEM((KB, 128, width), jnp.float32),
            pltpu.VMEM_SHARED((NPAD, width), jnp.float32),
            pltpu.SemaphoreType.DMA,
            pltpu.SemaphoreType.DMA,
            pltpu.SemaphoreType.DMA,
        ],
    )


def _deg_body(dst_hbm, zero_hbm, out_hbm, dst_v, ones_v, deg_sh, sem):
    c = lax.axis_index("c")
    s = lax.axis_index("s")
    wid = c * 16 + s
    pltpu.sync_copy(zero_hbm, deg_sh.at[pl.ds(s * RPT, RPT)])
    pltpu.sync_copy(dst_hbm.at[wid], dst_v)
    for i in range(8):
        ones_v[pl.ds(16 * i, 16)] = jnp.ones((16,), jnp.float32)
    plsc.subcore_barrier()

    def chunk(k, carry):
        descs = [pltpu.async_copy(ones_v, deg_sh.at[dst_v.at[8 * k + b]],
                                  sem, add=True)
                 for b in range(8)]
        for d in descs:
            d.wait()
        return carry

    lax.fori_loop(0, EB // 8, chunk, 0)
    plsc.subcore_barrier()
    pltpu.sync_copy(deg_sh.at[pl.ds(s * RPT, RPT)],
                    out_hbm.at[c].at[pl.ds(s * RPT, RPT)])


_deg_call = pl.kernel(
    _deg_body,
    out_type=jax.ShapeDtypeStruct((2, NPAD), jnp.float32),
    mesh=_mesh,
    compiler_params=pltpu.CompilerParams(use_tc_tiling_on_sc=False),
    scratch_types=[
        pltpu.VMEM((EB, 128), jnp.int32),
        pltpu.VMEM((128,), jnp.float32),
        pltpu.VMEM_SHARED((NPAD,), jnp.float32),
        pltpu.SemaphoreType.DMA,
    ],
)

_GRID = 8
_BR = NPAD // _GRID


def _tc1_body(degp_ref, x_ref, w1_ref, y1_ref, dinv_ref):
    deg = degp_ref[:, 0:1] + degp_ref[:, 1:2] + 1.0
    dinv = lax.rsqrt(deg)
    xw = jnp.dot(x_ref[...], w1_ref[...], preferred_element_type=jnp.float32)
    y1_ref[...] = xw * dinv
    dinv_ref[...] = dinv


def _tc2_body(agg_ref, y1_ref, dinv_ref, b1_ref, w2_ref, y2_ref):
    h = (agg_ref[0] + agg_ref[1] + y1_ref[...]) * dinv_ref[...] + b1_ref[...]
    h = jnp.maximum(h, 0.0)
    y2_ref[...] = jnp.dot(h, w2_ref[...],
                          preferred_element_type=jnp.float32) * dinv_ref[...]


def _tc3_body(agg_ref, y2_ref, dinv_ref, b2_ref, out_ref):
    z = (agg_ref[0] + agg_ref[1] + y2_ref[...]) * dinv_ref[...] + b2_ref[...]
    z = jnp.maximum(z, 0.0)
    col = lax.broadcasted_iota(jnp.int32, z.shape, 1)
    zm = jnp.where(col < NCLS, z, -jnp.inf)
    m = jnp.max(zm, axis=1, keepdims=True)
    e = jnp.where(col < NCLS, jnp.exp(z - m), 0.0)
    lse = jnp.log(jnp.sum(e, axis=1, keepdims=True))
    out_ref[...] = z - m - lse


def _row_spec(width):
    return pl.BlockSpec((_BR, width), lambda i: (i, 0))


def _agg_spec(width):
    return pl.BlockSpec((2, _BR, width), lambda i: (0, i, 0))


_tc1_call = pl.pallas_call(
    _tc1_body,
    grid=(_GRID,),
    in_specs=[_row_spec(2),
              _row_spec(DF),
              pl.BlockSpec((DF, DH), lambda i: (0, 0))],
    out_specs=[_row_spec(DH), _row_spec(1)],
    out_shape=[jax.ShapeDtypeStruct((NPAD, DH), jnp.float32),
               jax.ShapeDtypeStruct((NPAD, 1), jnp.float32)],
)

_tc2_call = pl.pallas_call(
    _tc2_body,
    grid=(_GRID,),
    in_specs=[_agg_spec(DH),
              _row_spec(DH),
              _row_spec(1),
              pl.BlockSpec((1, DH), lambda i: (0, 0)),
              pl.BlockSpec((DH, 8), lambda i: (0, 0))],
    out_specs=_row_spec(8),
    out_shape=jax.ShapeDtypeStruct((NPAD, 8), jnp.float32),
)

_tc3_call = pl.pallas_call(
    _tc3_body,
    grid=(_GRID,),
    in_specs=[_agg_spec(8),
              _row_spec(8),
              _row_spec(1),
              pl.BlockSpec((1, 8), lambda i: (0, 0))],
    out_specs=_row_spec(8),
    out_shape=jax.ShapeDtypeStruct((NPAD, 8), jnp.float32),
)

_agg_call16 = _make_agg(DH)
_agg_call8 = _make_agg(8)


@jax.jit
def kernel(x, edge_index, W1, b1, W2, b2):
    epad = jnp.full((NW * EPT - E,), N, dtype=jnp.int32)
    srcp = jnp.concatenate([edge_index[0], epad]).reshape(NW, EB, 128)
    dstp = jnp.concatenate([edge_index[1], epad]).reshape(NW, EB, 128)
    x_pad = jnp.concatenate(
        [x, jnp.zeros((NPAD - N, DF), jnp.float32)], axis=0)
    w2p = jnp.concatenate([W2, jnp.zeros((DH, 1), jnp.float32)], axis=1)
    b2p = jnp.concatenate([b2, jnp.zeros((1,), jnp.float32)]).reshape(1, 8)

    degp = _deg_call(dstp, jnp.zeros((RPT,), jnp.float32))
    y1, dinv = _tc1_call(degp.T, x_pad, W1)
    agg1 = _agg_call16(y1, srcp, dstp, jnp.zeros((RPT, DH), jnp.float32))
    y2 = _tc2_call(agg1, y1, dinv, b1.reshape(1, DH), w2p)
    agg2 = _agg_call8(y2, srcp, dstp, jnp.zeros((RPT, 8), jnp.float32))
    out = _tc3_call(agg2, y2, dinv, b2p)
    return out[:N, :NCLS]


# speedup vs baseline: 60.1827x; 1.6160x over previous
import jax
import jax.numpy as jnp
from jax import lax
from jax.experimental import pallas as pl
from jax.experimental.pallas import tpu as pltpu
from jax.experimental.pallas import tpu_sc as plsc

N = 10000
E = 320000
DF = 128
DH = 16
NCLS = 7

NPAD = 10240
RPT = NPAD // 16
NW = 32
EPT = 10240
EB = EPT // 128
KB = 8
NPAIR = EB // (2 * KB)

_mesh = plsc.VectorSubcoreMesh(core_axis_name="c", subcore_axis_name="s")


def _make_agg(width):
    def body(y_hbm, src_hbm, dst_hbm, zero_hbm, out_hbm,
             src_v, dst_v, rows0_v, rows1_v, agg_sh,
             sem_g0, sem_g1, sem_s):
        c = lax.axis_index("c")
        s = lax.axis_index("s")
        wid = c * 16 + s
        pltpu.sync_copy(zero_hbm, agg_sh.at[pl.ds(s * RPT, RPT)])
        pltpu.sync_copy(src_hbm.at[wid], src_v)
        pltpu.sync_copy(dst_hbm.at[wid], dst_v)
        plsc.subcore_barrier()

        def pair(k, carry):
            e0 = (2 * k) * KB
            e1 = (2 * k + 1) * KB
            g0 = [pltpu.async_copy(y_hbm.at[src_v.at[e0 + b]],
                                   rows0_v.at[b], sem_g0)
                  for b in range(KB)]
            g1 = [pltpu.async_copy(y_hbm.at[src_v.at[e1 + b]],
                                   rows1_v.at[b], sem_g1)
                  for b in range(KB)]
            for d in g0:
                d.wait()
            s0 = [pltpu.async_copy(rows0_v.at[b],
                                   agg_sh.at[dst_v.at[e0 + b]],
                                   sem_s, add=True)
                  for b in range(KB)]
            for d in g1:
                d.wait()
            s1 = [pltpu.async_copy(rows1_v.at[b],
                                   agg_sh.at[dst_v.at[e1 + b]],
                                   sem_s, add=True)
                  for b in range(KB)]
            for d in s0 + s1:
                d.wait()
            return carry

        lax.fori_loop(0, NPAIR, pair, 0)
        plsc.subcore_barrier()
        pltpu.sync_copy(agg_sh.at[pl.ds(s * RPT, RPT)],
                        out_hbm.at[c].at[pl.ds(s * RPT, RPT)])

    return pl.kernel(
        body,
        out_type=jax.ShapeDtypeStruct((2, NPAD, width), jnp.float32),
        mesh=_mesh,
        compiler_params=pltpu.CompilerParams(use_tc_tiling_on_sc=False),
        scratch_types=[
            pltpu.VMEM((EB, 128), jnp.int32),
            pltpu.VMEM((EB, 128), jnp.int32),
            pltpu.VMEM((KB, 128, width), jnp.float32),
            pltpu.VMEM((KB, 128, width), jnp.float32),
            pltpu.VMEM_SHARED((NPAD, width), jnp.float32),
            pltpu.SemaphoreType.DMA,
            pltpu.SemaphoreType.DMA,
            pltpu.SemaphoreType.DMA,
        ],
    )


def _deg_body(dst_hbm, zero_hbm, out_hbm, dst_v, ones_v, deg_sh, sem):
    c = lax.axis_index("c")
    s = lax.axis_index("s")
    wid = c * 16 + s
    pltpu.sync_copy(zero_hbm, deg_sh.at[pl.ds(s * RPT, RPT)])
    pltpu.sync_copy(dst_hbm.at[wid], dst_v)
    for i in range(8):
        ones_v[pl.ds(16 * i, 16)] = jnp.ones((16,), jnp.float32)
    plsc.subcore_barrier()

    def chunk(k, carry):
        descs = [pltpu.async_copy(ones_v, deg_sh.at[dst_v.at[8 * k + b]],
                                  sem, add=True)
                 for b in range(8)]
        for d in descs:
            d.wait()
        return carry

    lax.fori_loop(0, EB // 8, chunk, 0)
    plsc.subcore_barrier()
    pltpu.sync_copy(deg_sh.at[pl.ds(s * RPT, RPT)],
                    out_hbm.at[c].at[pl.ds(s * RPT, RPT)])


_deg_call = pl.kernel(
    _deg_body,
    out_type=jax.ShapeDtypeStruct((2, NPAD), jnp.float32),
    mesh=_mesh,
    compiler_params=pltpu.CompilerParams(use_tc_tiling_on_sc=False),
    scratch_types=[
        pltpu.VMEM((EB, 128), jnp.int32),
        pltpu.VMEM((128,), jnp.float32),
        pltpu.VMEM_SHARED((NPAD,), jnp.float32),
        pltpu.SemaphoreType.DMA,
    ],
)

_GRID = 8
_BR = NPAD // _GRID


def _tc1_body(degp_ref, x_ref, w1_ref, y1_ref, dinv_ref):
    deg = degp_ref[:, 0:1] + degp_ref[:, 1:2] + 1.0
    dinv = lax.rsqrt(deg)
    xw = jnp.dot(x_ref[...], w1_ref[...], preferred_element_type=jnp.float32)
    y1_ref[...] = xw * dinv
    dinv_ref[...] = dinv


def _tc2_body(agg_ref, y1_ref, dinv_ref, b1_ref, w2_ref, y2_ref):
    h = (agg_ref[0] + agg_ref[1] + y1_ref[...]) * dinv_ref[...] + b1_ref[...]
    h = jnp.maximum(h, 0.0)
    y2_ref[...] = jnp.dot(h, w2_ref[...],
                          preferred_element_type=jnp.float32) * dinv_ref[...]


def _tc3_body(agg_ref, y2_ref, dinv_ref, b2_ref, out_ref):
    z = (agg_ref[0] + agg_ref[1] + y2_ref[...]) * dinv_ref[...] + b2_ref[...]
    z = jnp.maximum(z, 0.0)
    col = lax.broadcasted_iota(jnp.int32, z.shape, 1)
    zm = jnp.where(col < NCLS, z, -jnp.inf)
    m = jnp.max(zm, axis=1, keepdims=True)
    e = jnp.where(col < NCLS, jnp.exp(z - m), 0.0)
    lse = jnp.log(jnp.sum(e, axis=1, keepdims=True))
    out_ref[...] = z - m - lse


def _row_spec(width):
    return pl.BlockSpec((_BR, width), lambda i: (i, 0))


def _agg_spec(width):
    return pl.BlockSpec((2, _BR, width), lambda i: (0, i, 0))


_tc1_call = pl.pallas_call(
    _tc1_body,
    grid=(_GRID,),
    in_specs=[_row_spec(2),
              _row_spec(DF),
              pl.BlockSpec((DF, DH), lambda i: (0, 0))],
    out_specs=[_row_spec(DH), _row_spec(1)],
    out_shape=[jax.ShapeDtypeStruct((NPAD, DH), jnp.float32),
               jax.ShapeDtypeStruct((NPAD, 1), jnp.float32)],
)

_tc2_call = pl.pallas_call(
    _tc2_body,
    grid=(_GRID,),
    in_specs=[_agg_spec(DH),
              _row_spec(DH),
              _row_spec(1),
              pl.BlockSpec((1, DH), lambda i: (0, 0)),
              pl.BlockSpec((DH, 8), lambda i: (0, 0))],
    out_specs=_row_spec(8),
    out_shape=jax.ShapeDtypeStruct((NPAD, 8), jnp.float32),
)

_tc3_call = pl.pallas_call(
    _tc3_body,
    grid=(_GRID,),
    in_specs=[_agg_spec(8),
              _row_spec(8),
              _row_spec(1),
              pl.BlockSpec((1, 8), lambda i: (0, 0))],
    out_specs=_row_spec(8),
    out_shape=jax.ShapeDtypeStruct((NPAD, 8), jnp.float32),
)

_agg_call16 = _make_agg(DH)
_agg_call8 = _make_agg(8)


@jax.jit
def kernel(x, edge_index, W1, b1, W2, b2):
    epad = N + jnp.arange(NW * EPT - E, dtype=jnp.int32) % (NPAD - N)
    srcp = jnp.concatenate([edge_index[0], epad]).reshape(NW, EB, 128)
    dstp = jnp.concatenate([edge_index[1], epad]).reshape(NW, EB, 128)
    x_pad = jnp.concatenate(
        [x, jnp.zeros((NPAD - N, DF), jnp.float32)], axis=0)
    w2p = jnp.concatenate([W2, jnp.zeros((DH, 1), jnp.float32)], axis=1)
    b2p = jnp.concatenate([b2, jnp.zeros((1,), jnp.float32)]).reshape(1, 8)

    degp = _deg_call(dstp, jnp.zeros((RPT,), jnp.float32))
    y1, dinv = _tc1_call(degp.T, x_pad, W1)
    agg1 = _agg_call16(y1, srcp, dstp, jnp.zeros((RPT, DH), jnp.float32))
    y2 = _tc2_call(agg1, y1, dinv, b1.reshape(1, DH), w2p)
    agg2 = _agg_call8(y2, srcp, dstp, jnp.zeros((RPT, 8), jnp.float32))
    out = _tc3_call(agg2, y2, dinv, b2p)
    return out[:N, :NCLS]


# speedup vs baseline: 76.7092x; 1.2746x over previous
import jax
import jax.numpy as jnp
from jax import lax
from jax.experimental import pallas as pl
from jax.experimental.pallas import tpu as pltpu
from jax.experimental.pallas import tpu_sc as plsc

N = 10000
E = 320000
DF = 128
DH = 16
NCLS = 7

NPAD = 10240
RPT = NPAD // 16
NW = 32
EPT = 10240
EB = EPT // 128
KB = 8
NPAIR = EB // (2 * KB)

_mesh = plsc.VectorSubcoreMesh(core_axis_name="c", subcore_axis_name="s")


def _make_agg(width):
    def body(y_hbm, src_hbm, dst_hbm, zero_hbm, out_hbm,
             src_v, dst_v, rows0_v, rows1_v, agg_sh,
             sem_g0, sem_g1, sem_s):
        c = lax.axis_index("c")
        s = lax.axis_index("s")
        wid = c * 16 + s
        pltpu.sync_copy(zero_hbm, agg_sh.at[pl.ds(s * RPT, RPT)])
        pltpu.sync_copy(src_hbm.at[wid], src_v)
        pltpu.sync_copy(dst_hbm.at[wid], dst_v)
        plsc.subcore_barrier()

        def pair(k, carry):
            e0 = (2 * k) * KB
            e1 = (2 * k + 1) * KB
            g0 = [pltpu.async_copy(y_hbm.at[src_v.at[e0 + b]],
                                   rows0_v.at[b], sem_g0)
                  for b in range(KB)]
            g1 = [pltpu.async_copy(y_hbm.at[src_v.at[e1 + b]],
                                   rows1_v.at[b], sem_g1)
                  for b in range(KB)]
            for d in g0:
                d.wait()
            s0 = [pltpu.async_copy(rows0_v.at[b],
                                   agg_sh.at[dst_v.at[e0 + b]],
                                   sem_s, add=True)
                  for b in range(KB)]
            for d in g1:
                d.wait()
            s1 = [pltpu.async_copy(rows1_v.at[b],
                                   agg_sh.at[dst_v.at[e1 + b]],
                                   sem_s, add=True)
                  for b in range(KB)]
            for d in s0 + s1:
                d.wait()
            return carry

        lax.fori_loop(0, NPAIR, pair, 0)
        plsc.subcore_barrier()
        pltpu.sync_copy(agg_sh.at[pl.ds(s * RPT, RPT)],
                        out_hbm.at[c].at[pl.ds(s * RPT, RPT)])

    return pl.kernel(
        body,
        out_type=jax.ShapeDtypeStruct((2, NPAD, width), jnp.float32),
        mesh=_mesh,
        compiler_params=pltpu.CompilerParams(use_tc_tiling_on_sc=False),
        scratch_types=[
            pltpu.VMEM((EB, 128), jnp.int32),
            pltpu.VMEM((EB, 128), jnp.int32),
            pltpu.VMEM((KB, 128, width), jnp.float32),
            pltpu.VMEM((KB, 128, width), jnp.float32),
            pltpu.VMEM_SHARED((NPAD, width), jnp.float32),
            pltpu.SemaphoreType.DMA,
            pltpu.SemaphoreType.DMA,
            pltpu.SemaphoreType.DMA,
        ],
    )


def _deg_body(dst_hbm, zero_hbm, out0_hbm, out1_hbm,
              dst_v, ones_v, deg_v, rep_v, deg_sh, sem):
    c = lax.axis_index("c")
    s = lax.axis_index("s")
    wid = c * 16 + s
    pltpu.sync_copy(zero_hbm, deg_sh.at[pl.ds(s * RPT, RPT)])
    pltpu.sync_copy(dst_hbm.at[wid], dst_v)
    for i in range(8):
        ones_v[pl.ds(16 * i, 16)] = jnp.ones((16,), jnp.float32)
    plsc.subcore_barrier()

    def chunk(k, carry):
        descs = [pltpu.async_copy(ones_v, deg_sh.at[dst_v.at[8 * k + b]],
                                  sem, add=True)
                 for b in range(8)]
        for d in descs:
            d.wait()
        return carry

    lax.fori_loop(0, EB // 8, chunk, 0)
    plsc.subcore_barrier()
    pltpu.sync_copy(deg_sh.at[pl.ds(s * RPT, RPT)], deg_v)

    def rep(t, carry):
        for u in range(4):
            j = t * 4 + u
            v = plsc.load_gather(deg_v, [jnp.full((16,), j, jnp.int32)])
            rep_v[pl.ds(pl.multiple_of(16 * j, 16), 16)] = v
        return carry

    lax.fori_loop(0, RPT // 4, rep, 0)

    @pl.when(c == 0)
    def _():
        pltpu.sync_copy(rep_v, out0_hbm.at[pl.ds(s * RPT * 16, RPT * 16)])

    @pl.when(c == 1)
    def _():
        pltpu.sync_copy(rep_v, out1_hbm.at[pl.ds(s * RPT * 16, RPT * 16)])


_deg_call = pl.kernel(
    _deg_body,
    out_type=[jax.ShapeDtypeStruct((NPAD * 16,), jnp.float32),
              jax.ShapeDtypeStruct((NPAD * 16,), jnp.float32)],
    mesh=_mesh,
    compiler_params=pltpu.CompilerParams(use_tc_tiling_on_sc=False,
                                         needs_layout_passes=False),
    scratch_types=[
        pltpu.VMEM((EB, 128), jnp.int32),
        pltpu.VMEM((128,), jnp.float32),
        pltpu.VMEM((RPT,), jnp.float32),
        pltpu.VMEM((RPT * 16,), jnp.float32),
        pltpu.VMEM_SHARED((NPAD,), jnp.float32),
        pltpu.SemaphoreType.DMA,
    ],
)

_GRID = 2
R16 = NPAD // 8
R8 = NPAD // 16


def _tc1_body(d0_ref, d1_ref, xp_ref, w1e_ref, y1_ref, dinv_ref):
    dinv = lax.rsqrt(d0_ref[...] + d1_ref[...] + 1.0)
    xw = jnp.dot(xp_ref[...], w1e_ref[...],
                 preferred_element_type=jnp.float32,
                 precision=lax.Precision.HIGHEST)
    y1_ref[...] = xw * dinv
    dinv_ref[...] = dinv


def _tc2_body(agg_ref, y1_ref, dinv_ref, b1_ref, w2e_ref, y2_ref):
    dinv = dinv_ref[...]
    h = (agg_ref[0] + agg_ref[1] + y1_ref[...]) * dinv + b1_ref[...]
    h = jnp.maximum(h, 0.0) * dinv
    y2_ref[...] = jnp.dot(h, w2e_ref[...],
                          preferred_element_type=jnp.float32,
                 precision=lax.Precision.HIGHEST)


def _tc3_body(agg_ref, y2_ref, dinv16_ref, k_ref, b2_ref, g_ref, h_ref,
              out_ref):
    dinv8 = jnp.dot(dinv16_ref[...], k_ref[...],
                    preferred_element_type=jnp.float32,
                 precision=lax.Precision.HIGHEST)
    z = (agg_ref[0] + agg_ref[1] + y2_ref[...]) * dinv8 + b2_ref[...]
    z = jnp.maximum(z, 0.0)
    m = jnp.max(z)
    lane = lax.broadcasted_iota(jnp.int32, z.shape, 1)
    e = jnp.where(lane % 8 < NCLS, jnp.exp(z - m), 0.0)
    ssum = jnp.dot(e, g_ref[...], preferred_element_type=jnp.float32,
                 precision=lax.Precision.HIGHEST)
    lse = jnp.dot(jnp.log(ssum), h_ref[...],
                  preferred_element_type=jnp.float32,
                 precision=lax.Precision.HIGHEST)
    out_ref[...] = z - m - lse


def _full(shape):
    return pl.BlockSpec(shape, lambda i: tuple(0 for _ in shape))


def _rows(rows, width):
    return pl.BlockSpec((rows // _GRID, width), lambda i: (i, 0))


def _arows(rows, width):
    return pl.BlockSpec((2, rows // _GRID, width), lambda i: (0, i, 0))


_tc1_call = pl.pallas_call(
    _tc1_body,
    grid=(_GRID,),
    in_specs=[_rows(R16, 128),
              _rows(R16, 128),
              _rows(R16, 1024),
              _full((1024, 128))],
    out_specs=[_rows(R16, 128), _rows(R16, 128)],
    out_shape=[jax.ShapeDtypeStruct((R16, 128), jnp.float32),
               jax.ShapeDtypeStruct((R16, 128), jnp.float32)],
)

_tc2_call = pl.pallas_call(
    _tc2_body,
    grid=(_GRID,),
    in_specs=[_arows(R8, 256),
              _rows(R8, 256),
              _rows(R8, 256),
              _full((1, 256)),
              _full((256, 128))],
    out_specs=_rows(R8, 128),
    out_shape=jax.ShapeDtypeStruct((R8, 128), jnp.float32),
)

_tc3_call = pl.pallas_call(
    _tc3_body,
    grid=(_GRID,),
    in_specs=[_arows(R8, 128),
              _rows(R8, 128),
              _rows(R8, 256),
              _full((256, 128)),
              _full((1, 128)),
              _full((128, 16)),
              _full((16, 128))],
    out_specs=_rows(R8, 128),
    out_shape=jax.ShapeDtypeStruct((R8, 128), jnp.float32),
)

_agg_call16 = _make_agg(DH)
_agg_call8 = _make_agg(8)


@jax.jit
def kernel(x, edge_index, W1, b1, W2, b2):
    epad = N + jnp.arange(NW * EPT - E, dtype=jnp.int32) % (NPAD - N)
    e_lin = edge_index.reshape(2 * E)
    srcp = jnp.concatenate([e_lin[:E], epad]).reshape(NW, EB, 128)
    dstp = jnp.concatenate([e_lin[E:], epad]).reshape(NW, EB, 128)
    x_pad = jnp.concatenate(
        [x, jnp.zeros((NPAD - N, DF), jnp.float32)], axis=0)
    xp = x_pad.reshape(R16, 8 * DF)
    w2p = jnp.concatenate([W2, jnp.zeros((DH, 1), jnp.float32)], axis=1)
    eye8 = jnp.eye(8, dtype=jnp.float32)
    eye16 = jnp.eye(16, dtype=jnp.float32)
    w1e = jnp.kron(eye8, W1)
    w2e = jnp.kron(eye16, w2p)
    kpick = jnp.kron(eye16, jnp.zeros((DH, 8), jnp.float32).at[0].set(1.0))
    gsum = jnp.kron(eye16, jnp.ones((8, 1), jnp.float32))
    hbro = jnp.kron(eye16, jnp.ones((1, 8), jnp.float32))
    b1rep = jnp.tile(b1, 16).reshape(1, 256)
    b2rep = jnp.tile(jnp.concatenate(
        [b2, jnp.zeros((1,), jnp.float32)]), 16).reshape(1, 128)

    d0, d1 = _deg_call(dstp, jnp.zeros((RPT,), jnp.float32))
    y1p, dinv16 = _tc1_call(d0.reshape(R16, 128), d1.reshape(R16, 128),
                            xp, w1e)
    agg1 = _agg_call16(y1p.reshape(NPAD, DH), srcp, dstp,
                       jnp.zeros((RPT, DH), jnp.float32))
    dinv16v = dinv16.reshape(R8, 256)
    y2p = _tc2_call(agg1.reshape(2, R8, 256), y1p.reshape(R8, 256),
                    dinv16v, b1rep, w2e)
    agg2 = _agg_call8(y2p.reshape(NPAD, 8), srcp, dstp,
                      jnp.zeros((RPT, 8), jnp.float32))
    lsp = _tc3_call(agg2.reshape(2, R8, 128), y2p, dinv16v,
                    kpick, b2rep, gsum, hbro)
    return lsp.reshape(NPAD, 8)[:N, :NCLS]


# speedup vs baseline: 78.2957x; 1.0207x over previous
import jax
import jax.numpy as jnp
from jax import lax
from jax.experimental import pallas as pl
from jax.experimental.pallas import tpu as pltpu
from jax.experimental.pallas import tpu_sc as plsc

N = 10000
E = 320000
DF = 128
DH = 16
NCLS = 7

NPAD = 10240
RPT = NPAD // 16
NW = 32
EPT = 10240
EB = EPT // 128
KB = 8
NPAIR = EB // (2 * KB)

_mesh = plsc.VectorSubcoreMesh(core_axis_name="c", subcore_axis_name="s")


def _make_agg(width):
    def body(y_hbm, src_hbm, dst_hbm, zero_hbm, out_hbm,
             src_v, dst_v, rows0_v, rows1_v, agg_sh,
             sem_g0, sem_g1, sem_s):
        c = lax.axis_index("c")
        s = lax.axis_index("s")
        wid = c * 16 + s
        pltpu.sync_copy(zero_hbm, agg_sh.at[pl.ds(s * RPT, RPT)])
        pltpu.sync_copy(src_hbm.at[wid], src_v)
        pltpu.sync_copy(dst_hbm.at[wid], dst_v)
        plsc.subcore_barrier()

        def pair(k, carry):
            e0 = (2 * k) * KB
            e1 = (2 * k + 1) * KB
            g0 = [pltpu.async_copy(y_hbm.at[src_v.at[e0 + b]],
                                   rows0_v.at[b], sem_g0)
                  for b in range(KB)]
            g1 = [pltpu.async_copy(y_hbm.at[src_v.at[e1 + b]],
                                   rows1_v.at[b], sem_g1)
                  for b in range(KB)]
            for d in g0:
                d.wait()
            s0 = [pltpu.async_copy(rows0_v.at[b],
                                   agg_sh.at[dst_v.at[e0 + b]],
                                   sem_s, add=True)
                  for b in range(KB)]
            for d in g1:
                d.wait()
            s1 = [pltpu.async_copy(rows1_v.at[b],
                                   agg_sh.at[dst_v.at[e1 + b]],
                                   sem_s, add=True)
                  for b in range(KB)]
            for d in s0 + s1:
                d.wait()
            return carry

        lax.fori_loop(0, NPAIR, pair, 0)
        plsc.subcore_barrier()
        pltpu.sync_copy(agg_sh.at[pl.ds(s * RPT, RPT)],
                        out_hbm.at[c].at[pl.ds(s * RPT, RPT)])

    return pl.kernel(
        body,
        out_type=jax.ShapeDtypeStruct((2, NPAD, width), jnp.float32),
        mesh=_mesh,
        compiler_params=pltpu.CompilerParams(use_tc_tiling_on_sc=False),
        scratch_types=[
            pltpu.VMEM((EB, 128), jnp.int32),
            pltpu.VMEM((EB, 128), jnp.int32),
            pltpu.VMEM((KB, 128, width), jnp.float32),
            pltpu.VMEM((KB, 128, width), jnp.float32),
            pltpu.VMEM_SHARED((NPAD, width), jnp.float32),
            pltpu.SemaphoreType.DMA,
            pltpu.SemaphoreType.DMA,
            pltpu.SemaphoreType.DMA,
        ],
    )


def _deg_body(dst_hbm, zero_hbm, out0_hbm, out1_hbm,
              dst_v, ones_v, deg_v, rep_v, deg_sh, sem):
    c = lax.axis_index("c")
    s = lax.axis_index("s")
    wid = c * 16 + s
    pltpu.sync_copy(zero_hbm, deg_sh.at[pl.ds(s * RPT, RPT)])
    pltpu.sync_copy(dst_hbm.at[wid], dst_v)
    for i in range(8):
        ones_v[pl.ds(16 * i, 16)] = jnp.ones((16,), jnp.float32)
    plsc.subcore_barrier()

    def chunk(k, carry):
        descs = [pltpu.async_copy(ones_v, deg_sh.at[dst_v.at[8 * k + b]],
                                  sem, add=True)
                 for b in range(8)]
        for d in descs:
            d.wait()
        return carry

    lax.fori_loop(0, EB // 8, chunk, 0)
    plsc.subcore_barrier()
    pltpu.sync_copy(deg_sh.at[pl.ds(s * RPT, RPT)], deg_v)

    def rep(t, carry):
        for u in range(4):
            j = t * 4 + u
            v = plsc.load_gather(deg_v, [jnp.full((16,), j, jnp.int32)])
            rep_v[pl.ds(pl.multiple_of(16 * j, 16), 16)] = v
        return carry

    lax.fori_loop(0, RPT // 4, rep, 0)

    @pl.when(c == 0)
    def _():
        pltpu.sync_copy(rep_v, out0_hbm.at[pl.ds(s * RPT * 16, RPT * 16)])

    @pl.when(c == 1)
    def _():
        pltpu.sync_copy(rep_v, out1_hbm.at[pl.ds(s * RPT * 16, RPT * 16)])


_deg_call = pl.kernel(
    _deg_body,
    out_type=[jax.ShapeDtypeStruct((NPAD * 16,), jnp.float32),
              jax.ShapeDtypeStruct((NPAD * 16,), jnp.float32)],
    mesh=_mesh,
    compiler_params=pltpu.CompilerParams(use_tc_tiling_on_sc=False,
                                         needs_layout_passes=False),
    scratch_types=[
        pltpu.VMEM((EB, 128), jnp.int32),
        pltpu.VMEM((128,), jnp.float32),
        pltpu.VMEM((RPT,), jnp.float32),
        pltpu.VMEM((RPT * 16,), jnp.float32),
        pltpu.VMEM_SHARED((NPAD,), jnp.float32),
        pltpu.SemaphoreType.DMA,
    ],
)

_GRID = 2
R16 = NPAD // 8
R8 = NPAD // 16


def _prep_body(ei_ref, src_ref, dst_ref):
    src_ref[...] = ei_ref[0, :]
    dst_ref[...] = ei_ref[1, :]


_prep_call = pl.pallas_call(
    _prep_body,
    out_shape=[jax.ShapeDtypeStruct((E,), jnp.int32),
               jax.ShapeDtypeStruct((E,), jnp.int32)],
)


def _tc1_body(d0_ref, d1_ref, xp_ref, w1e_ref, y1_ref, dinv_ref):
    dinv = lax.rsqrt(d0_ref[...] + d1_ref[...] + 1.0)
    xw = jnp.dot(xp_ref[...], w1e_ref[...],
                 preferred_element_type=jnp.float32,
                 precision=lax.Precision.HIGHEST)
    y1_ref[...] = xw * dinv
    dinv_ref[...] = dinv


def _tc2_body(agg_ref, y1_ref, dinv_ref, b1_ref, w2e_ref, y2_ref):
    dinv = dinv_ref[...]
    h = (agg_ref[0] + agg_ref[1] + y1_ref[...]) * dinv + b1_ref[...]
    h = jnp.maximum(h, 0.0) * dinv
    y2_ref[...] = jnp.dot(h, w2e_ref[...],
                          preferred_element_type=jnp.float32,
                 precision=lax.Precision.HIGHEST)


def _tc3_body(agg_ref, y2_ref, dinv16_ref, k_ref, b2_ref, g_ref, h_ref,
              out_ref):
    dinv8 = jnp.dot(dinv16_ref[...], k_ref[...],
                    preferred_element_type=jnp.float32,
                 precision=lax.Precision.HIGHEST)
    z = (agg_ref[0] + agg_ref[1] + y2_ref[...]) * dinv8 + b2_ref[...]
    z = jnp.maximum(z, 0.0)
    m = jnp.max(z)
    lane = lax.broadcasted_iota(jnp.int32, z.shape, 1)
    e = jnp.where(lane % 8 < NCLS, jnp.exp(z - m), 0.0)
    ssum = jnp.dot(e, g_ref[...], preferred_element_type=jnp.float32,
                 precision=lax.Precision.HIGHEST)
    lse = jnp.dot(jnp.log(ssum), h_ref[...],
                  preferred_element_type=jnp.float32,
                 precision=lax.Precision.HIGHEST)
    out_ref[...] = z - m - lse


def _full(shape):
    return pl.BlockSpec(shape, lambda i: tuple(0 for _ in shape))


def _rows(rows, width):
    return pl.BlockSpec((rows // _GRID, width), lambda i: (i, 0))


def _arows(rows, width):
    return pl.BlockSpec((2, rows // _GRID, width), lambda i: (0, i, 0))


_tc1_call = pl.pallas_call(
    _tc1_body,
    grid=(_GRID,),
    in_specs=[_rows(R16, 128),
              _rows(R16, 128),
              _rows(R16, 1024),
              _full((1024, 128))],
    out_specs=[_rows(R16, 128), _rows(R16, 128)],
    out_shape=[jax.ShapeDtypeStruct((R16, 128), jnp.float32),
               jax.ShapeDtypeStruct((R16, 128), jnp.float32)],
)

_tc2_call = pl.pallas_call(
    _tc2_body,
    grid=(_GRID,),
    in_specs=[_arows(R8, 256),
              _rows(R8, 256),
              _rows(R8, 256),
              _full((1, 256)),
              _full((256, 128))],
    out_specs=_rows(R8, 128),
    out_shape=jax.ShapeDtypeStruct((R8, 128), jnp.float32),
)

_tc3_call = pl.pallas_call(
    _tc3_body,
    grid=(_GRID,),
    in_specs=[_arows(R8, 128),
              _rows(R8, 128),
              _rows(R8, 256),
              _full((256, 128)),
              _full((1, 128)),
              _full((128, 16)),
              _full((16, 128))],
    out_specs=_rows(R8, 128),
    out_shape=jax.ShapeDtypeStruct((R8, 128), jnp.float32),
)

_agg_call16 = _make_agg(DH)
_agg_call8 = _make_agg(8)


@jax.jit
def kernel(x, edge_index, W1, b1, W2, b2):
    epad = N + jnp.arange(NW * EPT - E, dtype=jnp.int32) % (NPAD - N)
    src_f, dst_f = _prep_call(edge_index)
    srcp = jnp.concatenate([src_f, epad]).reshape(NW, EB, 128)
    dstp = jnp.concatenate([dst_f, epad]).reshape(NW, EB, 128)
    x_pad = jnp.concatenate(
        [x, jnp.zeros((NPAD - N, DF), jnp.float32)], axis=0)
    xp = x_pad.reshape(R16, 8 * DF)
    w2p = jnp.concatenate([W2, jnp.zeros((DH, 1), jnp.float32)], axis=1)
    eye8 = jnp.eye(8, dtype=jnp.float32)
    eye16 = jnp.eye(16, dtype=jnp.float32)
    w1e = jnp.kron(eye8, W1)
    w2e = jnp.kron(eye16, w2p)
    kpick = jnp.kron(eye16, jnp.zeros((DH, 8), jnp.float32).at[0].set(1.0))
    gsum = jnp.kron(eye16, jnp.ones((8, 1), jnp.float32))
    hbro = jnp.kron(eye16, jnp.ones((1, 8), jnp.float32))
    b1rep = jnp.tile(b1, 16).reshape(1, 256)
    b2rep = jnp.tile(jnp.concatenate(
        [b2, jnp.zeros((1,), jnp.float32)]), 16).reshape(1, 128)

    d0, d1 = _deg_call(dstp, jnp.zeros((RPT,), jnp.float32))
    y1p, dinv16 = _tc1_call(d0.reshape(R16, 128), d1.reshape(R16, 128),
                            xp, w1e)
    agg1 = _agg_call16(y1p.reshape(NPAD, DH), srcp, dstp,
                       jnp.zeros((RPT, DH), jnp.float32))
    dinv16v = dinv16.reshape(R8, 256)
    y2p = _tc2_call(agg1.reshape(2, R8, 256), y1p.reshape(R8, 256),
                    dinv16v, b1rep, w2e)
    agg2 = _agg_call8(y2p.reshape(NPAD, 8), srcp, dstp,
                      jnp.zeros((RPT, 8), jnp.float32))
    lsp = _tc3_call(agg2.reshape(2, R8, 128), y2p, dinv16v,
                    kpick, b2rep, gsum, hbro)
    gidx = (8 * jnp.arange(N, dtype=jnp.int32))[:, None] \
        + jnp.arange(NCLS, dtype=jnp.int32)[None, :]
    return lsp.reshape(NPAD * 8)[gidx]


# speedup vs baseline: 80.7498x; 1.0313x over previous
import jax
import jax.numpy as jnp
from jax import lax
from jax.experimental import pallas as pl
from jax.experimental.pallas import tpu as pltpu
from jax.experimental.pallas import tpu_sc as plsc

N = 10000
E = 320000
DF = 128
DH = 16
NCLS = 7

NPAD = 10240
RPT = NPAD // 16
NW = 32
EPT = 10240
EB = EPT // 128
KB = 8
NPAIR = EB // (2 * KB)

_mesh = plsc.VectorSubcoreMesh(core_axis_name="c", subcore_axis_name="s")


def _make_agg(width):
    def body(y_hbm, src_hbm, dst_hbm, zero_hbm, out_hbm,
             src_v, dst_v, rows0_v, rows1_v, agg_sh,
             sem_g0, sem_g1, sem_s):
        c = lax.axis_index("c")
        s = lax.axis_index("s")
        wid = c * 16 + s
        pltpu.sync_copy(zero_hbm, agg_sh.at[pl.ds(s * RPT, RPT)])
        pltpu.sync_copy(src_hbm.at[wid], src_v)
        pltpu.sync_copy(dst_hbm.at[wid], dst_v)
        plsc.subcore_barrier()

        def pair(k, carry):
            e0 = (2 * k) * KB
            e1 = (2 * k + 1) * KB
            g0 = [pltpu.async_copy(y_hbm.at[src_v.at[e0 + b]],
                                   rows0_v.at[b], sem_g0)
                  for b in range(KB)]
            g1 = [pltpu.async_copy(y_hbm.at[src_v.at[e1 + b]],
                                   rows1_v.at[b], sem_g1)
                  for b in range(KB)]
            for d in g0:
                d.wait()
            s0 = [pltpu.async_copy(rows0_v.at[b],
                                   agg_sh.at[dst_v.at[e0 + b]],
                                   sem_s, add=True)
                  for b in range(KB)]
            for d in g1:
                d.wait()
            s1 = [pltpu.async_copy(rows1_v.at[b],
                                   agg_sh.at[dst_v.at[e1 + b]],
                                   sem_s, add=True)
                  for b in range(KB)]
            for d in s0 + s1:
                d.wait()
            return carry

        lax.fori_loop(0, NPAIR, pair, 0)
        plsc.subcore_barrier()
        pltpu.sync_copy(agg_sh.at[pl.ds(s * RPT, RPT)],
                        out_hbm.at[c].at[pl.ds(s * RPT, RPT)])

    return pl.kernel(
        body,
        out_type=jax.ShapeDtypeStruct((2, NPAD, width), jnp.float32),
        mesh=_mesh,
        compiler_params=pltpu.CompilerParams(use_tc_tiling_on_sc=False),
        scratch_types=[
            pltpu.VMEM((EB, 128), jnp.int32),
            pltpu.VMEM((EB, 128), jnp.int32),
            pltpu.VMEM((KB, 128, width), jnp.float32),
            pltpu.VMEM((KB, 128, width), jnp.float32),
            pltpu.VMEM_SHARED((NPAD, width), jnp.float32),
            pltpu.SemaphoreType.DMA,
            pltpu.SemaphoreType.DMA,
            pltpu.SemaphoreType.DMA,
        ],
    )


def _deg_body(dst_hbm, zero_hbm, out0_hbm, out1_hbm,
              dst_v, ones_v, deg_v, rep_v, deg_sh, sem):
    c = lax.axis_index("c")
    s = lax.axis_index("s")
    wid = c * 16 + s
    pltpu.sync_copy(zero_hbm, deg_sh.at[pl.ds(s * RPT, RPT)])
    pltpu.sync_copy(dst_hbm.at[wid], dst_v)
    for i in range(8):
        ones_v[pl.ds(16 * i, 16)] = jnp.ones((16,), jnp.float32)
    plsc.subcore_barrier()

    def chunk(k, carry):
        descs = [pltpu.async_copy(ones_v, deg_sh.at[dst_v.at[8 * k + b]],
                                  sem, add=True)
                 for b in range(8)]
        for d in descs:
            d.wait()
        return carry

    lax.fori_loop(0, EB // 8, chunk, 0)
    plsc.subcore_barrier()
    pltpu.sync_copy(deg_sh.at[pl.ds(s * RPT, RPT)], deg_v)

    def rep(t, carry):
        for u in range(4):
            j = t * 4 + u
            v = plsc.load_gather(deg_v, [jnp.full((16,), j, jnp.int32)])
            rep_v[pl.ds(pl.multiple_of(16 * j, 16), 16)] = v
        return carry

    lax.fori_loop(0, RPT // 4, rep, 0)

    @pl.when(c == 0)
    def _():
        pltpu.sync_copy(rep_v, out0_hbm.at[pl.ds(s * RPT * 16, RPT * 16)])

    @pl.when(c == 1)
    def _():
        pltpu.sync_copy(rep_v, out1_hbm.at[pl.ds(s * RPT * 16, RPT * 16)])


_deg_call = pl.kernel(
    _deg_body,
    out_type=[jax.ShapeDtypeStruct((NPAD * 16,), jnp.float32),
              jax.ShapeDtypeStruct((NPAD * 16,), jnp.float32)],
    mesh=_mesh,
    compiler_params=pltpu.CompilerParams(use_tc_tiling_on_sc=False,
                                         needs_layout_passes=False),
    scratch_types=[
        pltpu.VMEM((EB, 128), jnp.int32),
        pltpu.VMEM((128,), jnp.float32),
        pltpu.VMEM((RPT,), jnp.float32),
        pltpu.VMEM((RPT * 16,), jnp.float32),
        pltpu.VMEM_SHARED((NPAD,), jnp.float32),
        pltpu.SemaphoreType.DMA,
    ],
)

_GRID = 2
R16 = NPAD // 8
R8 = NPAD // 16


def _prep_body(ei_ref, src_ref, dst_ref):
    src_ref[...] = ei_ref[0, :]
    dst_ref[...] = ei_ref[1, :]


_prep_call = pl.pallas_call(
    _prep_body,
    out_shape=[jax.ShapeDtypeStruct((E,), jnp.int32),
               jax.ShapeDtypeStruct((E,), jnp.int32)],
)


def _tc1_body(d0_ref, d1_ref, xp_ref, w1e_ref, y1_ref, dinv_ref):
    dinv = lax.rsqrt(d0_ref[...] + d1_ref[...] + 1.0)
    xw = jnp.dot(xp_ref[...], w1e_ref[...],
                 preferred_element_type=jnp.float32,
                 precision=lax.Precision.DEFAULT)
    y1_ref[...] = xw * dinv
    dinv_ref[...] = dinv


def _tc2_body(agg_ref, y1_ref, dinv_ref, b1_ref, w2e_ref, y2_ref):
    dinv = dinv_ref[...]
    h = (agg_ref[0] + agg_ref[1] + y1_ref[...]) * dinv + b1_ref[...]
    h = jnp.maximum(h, 0.0) * dinv
    y2_ref[...] = jnp.dot(h, w2e_ref[...],
                          preferred_element_type=jnp.float32,
                 precision=lax.Precision.DEFAULT)


def _tc3_body(agg_ref, y2_ref, dinv16_ref, k_ref, b2_ref, g_ref, h_ref,
              out_ref):
    dinv8 = jnp.dot(dinv16_ref[...], k_ref[...],
                    preferred_element_type=jnp.float32,
                 precision=lax.Precision.HIGHEST)
    z = (agg_ref[0] + agg_ref[1] + y2_ref[...]) * dinv8 + b2_ref[...]
    z = jnp.maximum(z, 0.0)
    m = jnp.max(z)
    lane = lax.broadcasted_iota(jnp.int32, z.shape, 1)
    e = jnp.where(lane % 8 < NCLS, jnp.exp(z - m), 0.0)
    ssum = jnp.dot(e, g_ref[...], preferred_element_type=jnp.float32,
                 precision=lax.Precision.HIGHEST)
    lse = jnp.dot(jnp.log(ssum), h_ref[...],
                  preferred_element_type=jnp.float32,
                 precision=lax.Precision.HIGHEST)
    out_ref[...] = z - m - lse


def _full(shape):
    return pl.BlockSpec(shape, lambda i: tuple(0 for _ in shape))


def _rows(rows, width):
    return pl.BlockSpec((rows // _GRID, width), lambda i: (i, 0))


def _arows(rows, width):
    return pl.BlockSpec((2, rows // _GRID, width), lambda i: (0, i, 0))


_tc1_call = pl.pallas_call(
    _tc1_body,
    grid=(_GRID,),
    in_specs=[_rows(R16, 128),
              _rows(R16, 128),
              _rows(R16, 1024),
              _full((1024, 128))],
    out_specs=[_rows(R16, 128), _rows(R16, 128)],
    out_shape=[jax.ShapeDtypeStruct((R16, 128), jnp.float32),
               jax.ShapeDtypeStruct((R16, 128), jnp.float32)],
)

_tc2_call = pl.pallas_call(
    _tc2_body,
    grid=(_GRID,),
    in_specs=[_arows(R8, 256),
              _rows(R8, 256),
              _rows(R8, 256),
              _full((1, 256)),
              _full((256, 128))],
    out_specs=_rows(R8, 128),
    out_shape=jax.ShapeDtypeStruct((R8, 128), jnp.float32),
)

_tc3_call = pl.pallas_call(
    _tc3_body,
    grid=(_GRID,),
    in_specs=[_arows(R8, 128),
              _rows(R8, 128),
              _rows(R8, 256),
              _full((256, 128)),
              _full((1, 128)),
              _full((128, 16)),
              _full((16, 128))],
    out_specs=_rows(R8, 128),
    out_shape=jax.ShapeDtypeStruct((R8, 128), jnp.float32),
)

_agg_call16 = _make_agg(DH)
_agg_call8 = _make_agg(8)


@jax.jit
def kernel(x, edge_index, W1, b1, W2, b2):
    epad = N + jnp.arange(NW * EPT - E, dtype=jnp.int32) % (NPAD - N)
    src_f, dst_f = _prep_call(edge_index)
    srcp = jnp.concatenate([src_f, epad]).reshape(NW, EB, 128)
    dstp = jnp.concatenate([dst_f, epad]).reshape(NW, EB, 128)
    x_pad = jnp.concatenate(
        [x, jnp.zeros((NPAD - N, DF), jnp.float32)], axis=0)
    xp = x_pad.reshape(R16, 8 * DF)
    w2p = jnp.concatenate([W2, jnp.zeros((DH, 1), jnp.float32)], axis=1)
    eye8 = jnp.eye(8, dtype=jnp.float32)
    eye16 = jnp.eye(16, dtype=jnp.float32)
    w1e = jnp.kron(eye8, W1)
    w2e = jnp.kron(eye16, w2p)
    kpick = jnp.kron(eye16, jnp.zeros((DH, 8), jnp.float32).at[0].set(1.0))
    gsum = jnp.kron(eye16, jnp.ones((8, 1), jnp.float32))
    hbro = jnp.kron(eye16, jnp.ones((1, 8), jnp.float32))
    b1rep = jnp.tile(b1, 16).reshape(1, 256)
    b2rep = jnp.tile(jnp.concatenate(
        [b2, jnp.zeros((1,), jnp.float32)]), 16).reshape(1, 128)

    d0, d1 = _deg_call(dstp, jnp.zeros((RPT,), jnp.float32))
    y1p, dinv16 = _tc1_call(d0.reshape(R16, 128), d1.reshape(R16, 128),
                            xp, w1e)
    agg1 = _agg_call16(y1p.reshape(NPAD, DH), srcp, dstp,
                       jnp.zeros((RPT, DH), jnp.float32))
    dinv16v = dinv16.reshape(R8, 256)
    y2p = _tc2_call(agg1.reshape(2, R8, 256), y1p.reshape(R8, 256),
                    dinv16v, b1rep, w2e)
    agg2 = _agg_call8(y2p.reshape(NPAD, 8), srcp, dstp,
                      jnp.zeros((RPT, 8), jnp.float32))
    lsp = _tc3_call(agg2.reshape(2, R8, 128), y2p, dinv16v,
                    kpick, b2rep, gsum, hbro)
    gidx = (8 * jnp.arange(N, dtype=jnp.int32))[:, None] \
        + jnp.arange(NCLS, dtype=jnp.int32)[None, :]
    return lsp.reshape(NPAD * 8)[gidx]


# speedup vs baseline: 81.0085x; 1.0032x over previous
import jax
import jax.numpy as jnp
from jax import lax
from jax.experimental import pallas as pl
from jax.experimental.pallas import tpu as pltpu
from jax.experimental.pallas import tpu_sc as plsc

N = 10000
E = 320000
DF = 128
DH = 16
NCLS = 7

NPAD = 10240
RPT = NPAD // 16
NW = 32
EPT = 10240
EB = EPT // 128
KB = 10
NPAIR = EB // (2 * KB)

_mesh = plsc.VectorSubcoreMesh(core_axis_name="c", subcore_axis_name="s")


def _make_agg(width):
    def body(y_hbm, src_hbm, dst_hbm, zero_hbm, out_hbm,
             src_v, dst_v, rows0_v, rows1_v, agg_sh,
             sem_g0, sem_g1, sem_s):
        c = lax.axis_index("c")
        s = lax.axis_index("s")
        wid = c * 16 + s
        pltpu.sync_copy(zero_hbm, agg_sh.at[pl.ds(s * RPT, RPT)])
        pltpu.sync_copy(src_hbm.at[wid], src_v)
        pltpu.sync_copy(dst_hbm.at[wid], dst_v)
        plsc.subcore_barrier()

        def pair(k, carry):
            e0 = (2 * k) * KB
            e1 = (2 * k + 1) * KB
            g0 = [pltpu.async_copy(y_hbm.at[src_v.at[e0 + b]],
                                   rows0_v.at[b], sem_g0)
                  for b in range(KB)]
            g1 = [pltpu.async_copy(y_hbm.at[src_v.at[e1 + b]],
                                   rows1_v.at[b], sem_g1)
                  for b in range(KB)]
            for d in g0:
                d.wait()
            s0 = [pltpu.async_copy(rows0_v.at[b],
                                   agg_sh.at[dst_v.at[e0 + b]],
                                   sem_s, add=True)
                  for b in range(KB)]
            for d in g1:
                d.wait()
            s1 = [pltpu.async_copy(rows1_v.at[b],
                                   agg_sh.at[dst_v.at[e1 + b]],
                                   sem_s, add=True)
                  for b in range(KB)]
            for d in s0 + s1:
                d.wait()
            return carry

        lax.fori_loop(0, NPAIR, pair, 0)
        plsc.subcore_barrier()
        pltpu.sync_copy(agg_sh.at[pl.ds(s * RPT, RPT)],
                        out_hbm.at[c].at[pl.ds(s * RPT, RPT)])

    return pl.kernel(
        body,
        out_type=jax.ShapeDtypeStruct((2, NPAD, width), jnp.float32),
        mesh=_mesh,
        compiler_params=pltpu.CompilerParams(use_tc_tiling_on_sc=False),
        scratch_types=[
            pltpu.VMEM((EB, 128), jnp.int32),
            pltpu.VMEM((EB, 128), jnp.int32),
            pltpu.VMEM((KB, 128, width), jnp.float32),
            pltpu.VMEM((KB, 128, width), jnp.float32),
            pltpu.VMEM_SHARED((NPAD, width), jnp.float32),
            pltpu.SemaphoreType.DMA,
            pltpu.SemaphoreType.DMA,
            pltpu.SemaphoreType.DMA,
        ],
    )


def _deg_body(dst_hbm, zero_hbm, out0_hbm, out1_hbm,
              dst_v, ones_v, deg_v, rep_v, deg_sh, sem):
    c = lax.axis_index("c")
    s = lax.axis_index("s")
    wid = c * 16 + s
    pltpu.sync_copy(zero_hbm, deg_sh.at[pl.ds(s * RPT, RPT)])
    pltpu.sync_copy(dst_hbm.at[wid], dst_v)
    for i in range(8):
        ones_v[pl.ds(16 * i, 16)] = jnp.ones((16,), jnp.float32)
    plsc.subcore_barrier()

    def chunk(k, carry):
        descs = [pltpu.async_copy(ones_v, deg_sh.at[dst_v.at[8 * k + b]],
                                  sem, add=True)
                 for b in range(8)]
        for d in descs:
            d.wait()
        return carry

    lax.fori_loop(0, EB // 8, chunk, 0)
    plsc.subcore_barrier()
    pltpu.sync_copy(deg_sh.at[pl.ds(s * RPT, RPT)], deg_v)

    def rep(t, carry):
        for u in range(4):
            j = t * 4 + u
            v = plsc.load_gather(deg_v, [jnp.full((16,), j, jnp.int32)])
            rep_v[pl.ds(pl.multiple_of(16 * j, 16), 16)] = v
        return carry

    lax.fori_loop(0, RPT // 4, rep, 0)

    @pl.when(c == 0)
    def _():
        pltpu.sync_copy(rep_v, out0_hbm.at[pl.ds(s * RPT * 16, RPT * 16)])

    @pl.when(c == 1)
    def _():
        pltpu.sync_copy(rep_v, out1_hbm.at[pl.ds(s * RPT * 16, RPT * 16)])


_deg_call = pl.kernel(
    _deg_body,
    out_type=[jax.ShapeDtypeStruct((NPAD * 16,), jnp.float32),
              jax.ShapeDtypeStruct((NPAD * 16,), jnp.float32)],
    mesh=_mesh,
    compiler_params=pltpu.CompilerParams(use_tc_tiling_on_sc=False,
                                         needs_layout_passes=False),
    scratch_types=[
        pltpu.VMEM((EB, 128), jnp.int32),
        pltpu.VMEM((128,), jnp.float32),
        pltpu.VMEM((RPT,), jnp.float32),
        pltpu.VMEM((RPT * 16,), jnp.float32),
        pltpu.VMEM_SHARED((NPAD,), jnp.float32),
        pltpu.SemaphoreType.DMA,
    ],
)

_GRID = 2
R16 = NPAD // 8
R8 = NPAD // 16


def _prep_body(ei_ref, src_ref, dst_ref):
    src_ref[...] = ei_ref[0, :]
    dst_ref[...] = ei_ref[1, :]


_prep_call = pl.pallas_call(
    _prep_body,
    out_shape=[jax.ShapeDtypeStruct((E,), jnp.int32),
               jax.ShapeDtypeStruct((E,), jnp.int32)],
)


def _tc1_body(d0_ref, d1_ref, xp_ref, w1e_ref, y1_ref, dinv_ref):
    dinv = lax.rsqrt(d0_ref[...] + d1_ref[...] + 1.0)
    xw = jnp.dot(xp_ref[...], w1e_ref[...],
                 preferred_element_type=jnp.float32,
                 precision=lax.Precision.DEFAULT)
    y1_ref[...] = xw * dinv
    dinv_ref[...] = dinv


def _tc2_body(agg_ref, y1_ref, dinv_ref, b1_ref, w2e_ref, y2_ref):
    dinv = dinv_ref[...]
    h = (agg_ref[0] + agg_ref[1] + y1_ref[...]) * dinv + b1_ref[...]
    h = jnp.maximum(h, 0.0) * dinv
    y2_ref[...] = jnp.dot(h, w2e_ref[...],
                          preferred_element_type=jnp.float32,
                 precision=lax.Precision.DEFAULT)


def _tc3_body(agg_ref, y2_ref, dinv16_ref, k_ref, b2_ref, g_ref, h_ref,
              out_ref):
    dinv8 = jnp.dot(dinv16_ref[...], k_ref[...],
                    preferred_element_type=jnp.float32,
                 precision=lax.Precision.HIGHEST)
    z = (agg_ref[0] + agg_ref[1] + y2_ref[...]) * dinv8 + b2_ref[...]
    z = jnp.maximum(z, 0.0)
    m = jnp.max(z)
    lane = lax.broadcasted_iota(jnp.int32, z.shape, 1)
    e = jnp.where(lane % 8 < NCLS, jnp.exp(z - m), 0.0)
    ssum = jnp.dot(e, g_ref[...], preferred_element_type=jnp.float32,
                 precision=lax.Precision.HIGHEST)
    lse = jnp.dot(jnp.log(ssum), h_ref[...],
                  preferred_element_type=jnp.float32,
                 precision=lax.Precision.HIGHEST)
    out_ref[...] = z - m - lse


def _full(shape):
    return pl.BlockSpec(shape, lambda i: tuple(0 for _ in shape))


def _rows(rows, width):
    return pl.BlockSpec((rows // _GRID, width), lambda i: (i, 0))


def _arows(rows, width):
    return pl.BlockSpec((2, rows // _GRID, width), lambda i: (0, i, 0))


_tc1_call = pl.pallas_call(
    _tc1_body,
    grid=(_GRID,),
    in_specs=[_rows(R16, 128),
              _rows(R16, 128),
              _rows(R16, 1024),
              _full((1024, 128))],
    out_specs=[_rows(R16, 128), _rows(R16, 128)],
    out_shape=[jax.ShapeDtypeStruct((R16, 128), jnp.float32),
               jax.ShapeDtypeStruct((R16, 128), jnp.float32)],
)

_tc2_call = pl.pallas_call(
    _tc2_body,
    grid=(_GRID,),
    in_specs=[_arows(R8, 256),
              _rows(R8, 256),
              _rows(R8, 256),
              _full((1, 256)),
              _full((256, 128))],
    out_specs=_rows(R8, 128),
    out_shape=jax.ShapeDtypeStruct((R8, 128), jnp.float32),
)

_tc3_call = pl.pallas_call(
    _tc3_body,
    grid=(_GRID,),
    in_specs=[_arows(R8, 128),
              _rows(R8, 128),
              _rows(R8, 256),
              _full((256, 128)),
              _full((1, 128)),
              _full((128, 16)),
              _full((16, 128))],
    out_specs=_rows(R8, 128),
    out_shape=jax.ShapeDtypeStruct((R8, 128), jnp.float32),
)

_agg_call16 = _make_agg(DH)
_agg_call8 = _make_agg(8)


@jax.jit
def kernel(x, edge_index, W1, b1, W2, b2):
    epad = N + jnp.arange(NW * EPT - E, dtype=jnp.int32) % (NPAD - N)
    src_f, dst_f = _prep_call(edge_index)
    srcp = jnp.concatenate([src_f, epad]).reshape(NW, EB, 128)
    dstp = jnp.concatenate([dst_f, epad]).reshape(NW, EB, 128)
    x_pad = jnp.concatenate(
        [x, jnp.zeros((NPAD - N, DF), jnp.float32)], axis=0)
    xp = x_pad.reshape(R16, 8 * DF)
    w2p = jnp.concatenate([W2, jnp.zeros((DH, 1), jnp.float32)], axis=1)
    eye8 = jnp.eye(8, dtype=jnp.float32)
    eye16 = jnp.eye(16, dtype=jnp.float32)
    w1e = jnp.kron(eye8, W1)
    w2e = jnp.kron(eye16, w2p)
    kpick = jnp.kron(eye16, jnp.zeros((DH, 8), jnp.float32).at[0].set(1.0))
    gsum = jnp.kron(eye16, jnp.ones((8, 1), jnp.float32))
    hbro = jnp.kron(eye16, jnp.ones((1, 8), jnp.float32))
    b1rep = jnp.tile(b1, 16).reshape(1, 256)
    b2rep = jnp.tile(jnp.concatenate(
        [b2, jnp.zeros((1,), jnp.float32)]), 16).reshape(1, 128)

    d0, d1 = _deg_call(dstp, jnp.zeros((RPT,), jnp.float32))
    y1p, dinv16 = _tc1_call(d0.reshape(R16, 128), d1.reshape(R16, 128),
                            xp, w1e)
    agg1 = _agg_call16(y1p.reshape(NPAD, DH), srcp, dstp,
                       jnp.zeros((RPT, DH), jnp.float32))
    dinv16v = dinv16.reshape(R8, 256)
    y2p = _tc2_call(agg1.reshape(2, R8, 256), y1p.reshape(R8, 256),
                    dinv16v, b1rep, w2e)
    agg2 = _agg_call8(y2p.reshape(NPAD, 8), srcp, dstp,
                      jnp.zeros((RPT, 8), jnp.float32))
    lsp = _tc3_call(agg2.reshape(2, R8, 128), y2p, dinv16v,
                    kpick, b2rep, gsum, hbro)
    gidx = (8 * jnp.arange(N, dtype=jnp.int32))[:, None] \
        + jnp.arange(NCLS, dtype=jnp.int32)[None, :]
    return lsp.reshape(NPAD * 8)[gidx]


# speedup vs baseline: 86.9708x; 1.0736x over previous
import jax
import jax.numpy as jnp
from jax import lax
from jax.experimental import pallas as pl
from jax.experimental.pallas import tpu as pltpu
from jax.experimental.pallas import tpu_sc as plsc

N = 10000
E = 320000
DF = 128
DH = 16
NCLS = 7

NPAD = 10240
RPT = NPAD // 16
NW = 32
EPT = 10240
EB = EPT // 128
KB = 10
NPAIR = EB // (2 * KB)

_mesh = plsc.VectorSubcoreMesh(core_axis_name="c", subcore_axis_name="s")


def _make_agg(width):
    def body(y_hbm, src_hbm, dst_hbm, zero_hbm, out_hbm,
             src_v, dst_v, rows0_v, rows1_v, agg_sh,
             sem_g0, sem_g1, sem_s):
        c = lax.axis_index("c")
        s = lax.axis_index("s")
        wid = c * 16 + s
        pltpu.sync_copy(zero_hbm, agg_sh.at[pl.ds(s * RPT, RPT)])
        pltpu.sync_copy(src_hbm.at[wid], src_v)
        pltpu.sync_copy(dst_hbm.at[wid], dst_v)
        plsc.subcore_barrier()

        def pair(k, carry):
            e0 = (2 * k) * KB
            e1 = (2 * k + 1) * KB
            g0 = [pltpu.async_copy(y_hbm.at[src_v.at[e0 + b]],
                                   rows0_v.at[b], sem_g0)
                  for b in range(KB)]
            g1 = [pltpu.async_copy(y_hbm.at[src_v.at[e1 + b]],
                                   rows1_v.at[b], sem_g1)
                  for b in range(KB)]
            for d in g0:
                d.wait()
            s0 = [pltpu.async_copy(rows0_v.at[b],
                                   agg_sh.at[dst_v.at[e0 + b]],
                                   sem_s, add=True)
                  for b in range(KB)]
            for d in g1:
                d.wait()
            s1 = [pltpu.async_copy(rows1_v.at[b],
                                   agg_sh.at[dst_v.at[e1 + b]],
                                   sem_s, add=True)
                  for b in range(KB)]
            for d in s0 + s1:
                d.wait()
            return carry

        lax.fori_loop(0, NPAIR, pair, 0)
        plsc.subcore_barrier()
        pltpu.sync_copy(agg_sh.at[pl.ds(s * RPT, RPT)],
                        out_hbm.at[c].at[pl.ds(s * RPT, RPT)])

    return pl.kernel(
        body,
        out_type=jax.ShapeDtypeStruct((2, NPAD, width), jnp.float32),
        mesh=_mesh,
        compiler_params=pltpu.CompilerParams(use_tc_tiling_on_sc=False),
        scratch_types=[
            pltpu.VMEM((EB, 128), jnp.int32),
            pltpu.VMEM((EB, 128), jnp.int32),
            pltpu.VMEM((KB, 128, width), jnp.float32),
            pltpu.VMEM((KB, 128, width), jnp.float32),
            pltpu.VMEM_SHARED((NPAD, width), jnp.float32),
            pltpu.SemaphoreType.DMA,
            pltpu.SemaphoreType.DMA,
            pltpu.SemaphoreType.DMA,
        ],
    )


def _deg_body(dst_hbm, zero_hbm, out0_hbm, out1_hbm,
              dst_v, ones_v, deg_v, rep_v, deg_sh, sem):
    c = lax.axis_index("c")
    s = lax.axis_index("s")
    wid = c * 16 + s
    pltpu.sync_copy(zero_hbm, deg_sh.at[pl.ds(s * RPT, RPT)])
    pltpu.sync_copy(dst_hbm.at[wid], dst_v)
    for i in range(8):
        ones_v[pl.ds(16 * i, 16)] = jnp.ones((16,), jnp.float32)
    plsc.subcore_barrier()

    def chunk(k, carry):
        descs = [pltpu.async_copy(ones_v, deg_sh.at[dst_v.at[8 * k + b]],
                                  sem, add=True)
                 for b in range(8)]
        for d in descs:
            d.wait()
        return carry

    lax.fori_loop(0, EB // 8, chunk, 0)
    plsc.subcore_barrier()
    pltpu.sync_copy(deg_sh.at[pl.ds(s * RPT, RPT)], deg_v)

    def rep(t, carry):
        for u in range(4):
            j = t * 4 + u
            v = plsc.load_gather(deg_v, [jnp.full((16,), j, jnp.int32)])
            rep_v[pl.ds(pl.multiple_of(16 * j, 16), 16)] = v
        return carry

    lax.fori_loop(0, RPT // 4, rep, 0)

    @pl.when(c == 0)
    def _():
        pltpu.sync_copy(rep_v, out0_hbm.at[pl.ds(s * RPT * 16, RPT * 16)])

    @pl.when(c == 1)
    def _():
        pltpu.sync_copy(rep_v, out1_hbm.at[pl.ds(s * RPT * 16, RPT * 16)])


_deg_call = pl.kernel(
    _deg_body,
    out_type=[jax.ShapeDtypeStruct((NPAD * 16,), jnp.float32),
              jax.ShapeDtypeStruct((NPAD * 16,), jnp.float32)],
    mesh=_mesh,
    compiler_params=pltpu.CompilerParams(use_tc_tiling_on_sc=False,
                                         needs_layout_passes=False),
    scratch_types=[
        pltpu.VMEM((EB, 128), jnp.int32),
        pltpu.VMEM((128,), jnp.float32),
        pltpu.VMEM((RPT,), jnp.float32),
        pltpu.VMEM((RPT * 16,), jnp.float32),
        pltpu.VMEM_SHARED((NPAD,), jnp.float32),
        pltpu.SemaphoreType.DMA,
    ],
)

_GRID = 2
R16 = NPAD // 8
R8 = NPAD // 16


def _prep_body(ei_ref, src_ref, dst_ref):
    src_ref[...] = ei_ref[0, :]
    dst_ref[...] = ei_ref[1, :]


_prep_call = pl.pallas_call(
    _prep_body,
    out_shape=[jax.ShapeDtypeStruct((E,), jnp.int32),
               jax.ShapeDtypeStruct((E,), jnp.int32)],
)


def _tc1_body(d0_ref, d1_ref, xp_ref, w1e_ref, y1_ref, dinv_ref):
    dinv = lax.rsqrt(d0_ref[...] + d1_ref[...] + 1.0)
    xw = jnp.dot(xp_ref[...], w1e_ref[...],
                 preferred_element_type=jnp.float32,
                 precision=lax.Precision.DEFAULT)
    y1_ref[...] = xw * dinv
    dinv_ref[...] = dinv


def _tc2_body(agg_ref, y1_ref, dinv_ref, b1_ref, w2e_ref, y2_ref):
    dinv = dinv_ref[...]
    h = (agg_ref[0] + agg_ref[1] + y1_ref[...]) * dinv + b1_ref[...]
    h = jnp.maximum(h, 0.0) * dinv
    y2_ref[...] = jnp.dot(h, w2e_ref[...],
                          preferred_element_type=jnp.float32,
                 precision=lax.Precision.DEFAULT)


def _tc3_body(agg_ref, y2_ref, dinv16_ref, k_ref, b2_ref, g_ref, h_ref,
              out_ref):
    dinv8 = jnp.dot(dinv16_ref[...], k_ref[...],
                    preferred_element_type=jnp.float32,
                 precision=lax.Precision.HIGHEST)
    z = (agg_ref[0] + agg_ref[1] + y2_ref[...]) * dinv8 + b2_ref[...]
    z = jnp.maximum(z, 0.0)
    m = jnp.max(z)
    lane = lax.broadcasted_iota(jnp.int32, z.shape, 1)
    e = jnp.where(lane % 8 < NCLS, jnp.exp(z - m), 0.0)
    ssum = jnp.dot(e, g_ref[...], preferred_element_type=jnp.float32,
                 precision=lax.Precision.HIGHEST)
    lse = jnp.dot(jnp.log(ssum), h_ref[...],
                  preferred_element_type=jnp.float32,
                 precision=lax.Precision.HIGHEST)
    out_ref[...] = z - m - lse


def _full(shape):
    return pl.BlockSpec(shape, lambda i: tuple(0 for _ in shape))


def _rows(rows, width):
    return pl.BlockSpec((rows // _GRID, width), lambda i: (i, 0))


def _arows(rows, width):
    return pl.BlockSpec((2, rows // _GRID, width), lambda i: (0, i, 0))


_tc1_call = pl.pallas_call(
    _tc1_body,
    grid=(_GRID,),
    in_specs=[_rows(R16, 128),
              _rows(R16, 128),
              _rows(R16, 1024),
              _full((1024, 128))],
    out_specs=[_rows(R16, 128), _rows(R16, 128)],
    out_shape=[jax.ShapeDtypeStruct((R16, 128), jnp.float32),
               jax.ShapeDtypeStruct((R16, 128), jnp.float32)],
)

_tc2_call = pl.pallas_call(
    _tc2_body,
    grid=(_GRID,),
    in_specs=[_arows(R8, 256),
              _rows(R8, 256),
              _rows(R8, 256),
              _full((1, 256)),
              _full((256, 128))],
    out_specs=_rows(R8, 128),
    out_shape=jax.ShapeDtypeStruct((R8, 128), jnp.float32),
)

_tc3_call = pl.pallas_call(
    _tc3_body,
    grid=(_GRID,),
    in_specs=[_arows(R8, 128),
              _rows(R8, 128),
              _rows(R8, 256),
              _full((256, 128)),
              _full((1, 128)),
              _full((128, 16)),
              _full((16, 128))],
    out_specs=_rows(R8, 128),
    out_shape=jax.ShapeDtypeStruct((R8, 128), jnp.float32),
)

_agg_call16 = _make_agg(DH)
_agg_call8 = _make_agg(8)


@jax.jit
def kernel(x, edge_index, W1, b1, W2, b2):
    epad = N + jnp.arange(NW * EPT - E, dtype=jnp.int32) % (NPAD - N)
    src_f, dst_f = _prep_call(edge_index)
    srcp = jnp.concatenate([src_f, epad]).reshape(NW, EB, 128)
    dstp = jnp.concatenate([dst_f, epad]).reshape(NW, EB, 128)
    x_pad = jnp.concatenate(
        [x, jnp.zeros((NPAD - N, DF), jnp.float32)], axis=0)
    xp = x_pad.reshape(R16, 8 * DF)
    w2p = jnp.concatenate([W2, jnp.zeros((DH, 1), jnp.float32)], axis=1)
    eye8 = jnp.eye(8, dtype=jnp.float32)
    eye16 = jnp.eye(16, dtype=jnp.float32)
    w1e = jnp.kron(eye8, W1)
    w2e = jnp.kron(eye16, w2p)
    kpick = jnp.kron(eye16, jnp.zeros((DH, 8), jnp.float32).at[0].set(1.0))
    gsum = jnp.kron(eye16, jnp.ones((8, 1), jnp.float32))
    hbro = jnp.kron(eye16, jnp.ones((1, 8), jnp.float32))
    b1rep = jnp.tile(b1, 16).reshape(1, 256)
    b2rep = jnp.tile(jnp.concatenate(
        [b2, jnp.zeros((1,), jnp.float32)]), 16).reshape(1, 128)

    d0, d1 = _deg_call(dstp, jnp.zeros((RPT,), jnp.float32))
    y1p, dinv16 = _tc1_call(d0.reshape(R16, 128), d1.reshape(R16, 128),
                            xp, w1e)
    agg1 = _agg_call16(y1p.reshape(NPAD, DH), srcp, dstp,
                       jnp.zeros((RPT, DH), jnp.float32))
    dinv16v = dinv16.reshape(R8, 256)
    y2p = _tc2_call(agg1.reshape(2, R8, 256), y1p.reshape(R8, 256),
                    dinv16v, b1rep, w2e)
    agg2 = _agg_call8(y2p.reshape(NPAD, 8), srcp, dstp,
                      jnp.zeros((RPT, 8), jnp.float32))
    lsp = _tc3_call(agg2.reshape(2, R8, 128), y2p, dinv16v,
                    kpick, b2rep, gsum, hbro)
    return lsp.reshape(NPAD, 8)[:N, :NCLS]
